# baseline (speedup 1.0000x reference)
import functools
import math

import jax
import jax.numpy as jnp
from jax import lax
from jax.experimental import pallas as pl
from jax.experimental.pallas import tpu as pltpu
from jax.experimental.pallas import tpu_sc as plsc

_AVG_AGG = 32.0
_SCALE = 1.0 / math.sqrt(_AVG_AGG)
_N = 10000
_NC = 2
_NS = 16
_NW = _NC * _NS
_B = 128
_RING = 3
_ZB = 64


def _sc_scatter_partials(x, idx):
    e_total, d = x.shape
    assert d == 128
    n_chunks = e_total // _B
    assert n_chunks * _B == e_total
    base_n = n_chunks // _NW
    n_extra = n_chunks - base_n * _NW
    assert base_n % _RING == 0 and n_extra < _NW
    w_extra = _NW - n_extra
    rows_per_tile = (_N // _NS) // 8 * 8
    rows_rem = _N - rows_per_tile * _NS

    mesh = plsc.VectorSubcoreMesh(core_axis_name="c", subcore_axis_name="s")

    @functools.partial(
        pl.kernel,
        mesh=mesh,
        out_type=jax.ShapeDtypeStruct((_NC, _N, d), jnp.float32),
        scratch_types=(
            [pltpu.VMEM((_B,), jnp.int32) for _ in range(_RING)]
            + [pltpu.VMEM((_B, d), jnp.float32) for _ in range(_RING)]
            + [pltpu.VMEM_SHARED((_N, d), jnp.float32)]
            + [pltpu.SemaphoreType.DMA for _ in range(_RING + 1)]
        ),
    )
    def scatter_kernel(x_hbm, idx_hbm, out_hbm, *refs):
        idx_v = refs[:_RING]
        rows_v = refs[_RING:2 * _RING]
        acc_sh = refs[2 * _RING]
        sems = refs[2 * _RING + 1:3 * _RING + 1]
        sem_z = refs[3 * _RING + 1]

        cid = lax.axis_index("c")
        sid = lax.axis_index("s")
        wid = cid * _NS + sid
        cbase = base_n * wid + jnp.maximum(0, wid - w_extra)
        n_w = base_n + (wid >= w_extra).astype(jnp.int32)

        def start_loads(g, b):
            e0 = (cbase + g) * _B
            pltpu.async_copy(idx_hbm.at[pl.ds(e0, _B)], idx_v[b], sems[b])
            pltpu.async_copy(x_hbm.at[pl.ds(e0, _B)], rows_v[b], sems[b])

        def wait_loads(g, b):
            e0 = (cbase + g) * _B
            pltpu.make_async_copy(idx_hbm.at[pl.ds(e0, _B)],
                                  idx_v[b], sems[b]).wait()
            pltpu.make_async_copy(x_hbm.at[pl.ds(e0, _B)],
                                  rows_v[b], sems[b]).wait()

        zeros16 = jnp.zeros((16,), jnp.float32)

        def zero_row(i, _):
            for j in range(d // 16):
                rows_v[0][i, pl.ds(j * 16, 16)] = zeros16
            return 0

        lax.fori_loop(0, _ZB, zero_row, 0)

        zsrc = rows_v[0].at[pl.ds(0, _ZB)]
        zbase = sid * rows_per_tile
        full, rem = divmod(rows_per_tile, _ZB)
        zcopies = []
        for k in range(full):
            zcopies.append(pltpu.async_copy(
                zsrc, acc_sh.at[pl.ds(zbase + k * _ZB, _ZB)], sem_z))
        if rem:
            zcopies.append(pltpu.async_copy(
                rows_v[0].at[pl.ds(0, rem)],
                acc_sh.at[pl.ds(zbase + full * _ZB, rem)], sem_z))

        @pl.when(sid == 0)
        def _zero_tail():
            pltpu.async_copy(
                rows_v[0].at[pl.ds(0, rows_rem)],
                acc_sh.at[pl.ds(rows_per_tile * _NS, rows_rem)],
                sem_z).wait()

        for b in range(1, _RING):
            start_loads(b, b)
        for c in zcopies:
            c.wait()
        start_loads(0, 0)
        plsc.subcore_barrier()

        @pl.loop(0, base_n // _RING)
        def _ring(i):
            for b in range(_RING):
                g = i * _RING + b
                wait_loads(g, b)
                pltpu.sync_copy(rows_v[b], acc_sh.at[idx_v[b]], add=True)

                @pl.when(g + _RING < n_w)
                def _prefetch():
                    start_loads(g + _RING, b)

        @pl.when(n_w > base_n)
        def _extra_chunk():
            wait_loads(base_n, 0)
            pltpu.sync_copy(rows_v[0], acc_sh.at[idx_v[0]], add=True)

        plsc.subcore_barrier()

        obase = sid * rows_per_tile
        pltpu.sync_copy(
            acc_sh.at[pl.ds(obase, rows_per_tile)],
            out_hbm.at[cid, pl.ds(obase, rows_per_tile)],
        )

        @pl.when(sid == 0)
        def _write_tail():
            pltpu.sync_copy(
                acc_sh.at[pl.ds(rows_per_tile * _NS, rows_rem)],
                out_hbm.at[cid, pl.ds(rows_per_tile * _NS, rows_rem)],
            )

    return scatter_kernel(x, idx)


def _combine(p_ref, o_ref):
    o_ref[...] = (p_ref[0] + p_ref[1]) * _SCALE


def kernel(x, index):
    idx = index.astype(jnp.int32)
    partials = _sc_scatter_partials(x, idx)
    n, d = _N, x.shape[1]
    blk = 2000
    out = pl.pallas_call(
        _combine,
        grid=(n // blk,),
        in_specs=[pl.BlockSpec((_NC, blk, d), lambda i: (0, i, 0))],
        out_specs=pl.BlockSpec((blk, d), lambda i: (i, 0)),
        out_shape=jax.ShapeDtypeStruct((n, d), jnp.float32),
    )(partials)
    return out


# speedup vs baseline: 1.0247x; 1.0247x over previous
import functools
import math

import jax
import jax.numpy as jnp
from jax import lax
from jax.experimental import pallas as pl
from jax.experimental.pallas import tpu as pltpu
from jax.experimental.pallas import tpu_sc as plsc

_AVG_AGG = 32.0
_SCALE = 1.0 / math.sqrt(_AVG_AGG)
_N = 10000
_NC = 2
_NS = 16
_NW = _NC * _NS
_B = 80
_RING = 4
_ZB = 64


def _sc_scatter_partials(x, idx):
    e_total, d = x.shape
    assert d == 128
    assert e_total % (_NW * _B) == 0
    e_per_w = e_total // _NW
    n_iter = e_per_w // _B
    assert n_iter % _RING == 1
    rows_per_tile = (_N // _NS) // 8 * 8
    rows_rem = _N - rows_per_tile * _NS

    mesh = plsc.VectorSubcoreMesh(core_axis_name="c", subcore_axis_name="s")

    @functools.partial(
        pl.kernel,
        mesh=mesh,
        out_type=jax.ShapeDtypeStruct((_NC, _N, d), jnp.float32),
        scratch_types=(
            [pltpu.VMEM((_B,), jnp.int32) for _ in range(_RING)]
            + [pltpu.VMEM((_B, d), jnp.float32) for _ in range(_RING)]
            + [pltpu.VMEM((_ZB, d), jnp.float32)]
            + [pltpu.VMEM_SHARED((_N, d), jnp.float32)]
            + [pltpu.SemaphoreType.DMA for _ in range(_RING + 1)]
        ),
    )
    def scatter_kernel(x_hbm, idx_hbm, out_hbm, *refs):
        idx_v = refs[:_RING]
        rows_v = refs[_RING:2 * _RING]
        zero_v = refs[2 * _RING]
        acc_sh = refs[2 * _RING + 1]
        sems = refs[2 * _RING + 2:3 * _RING + 2]
        sem_z = refs[3 * _RING + 2]

        cid = lax.axis_index("c")
        sid = lax.axis_index("s")
        wid = cid * _NS + sid
        ebase = wid * e_per_w

        def start_loads(g, b):
            pltpu.async_copy(idx_hbm.at[pl.ds(ebase + g * _B, _B)],
                             idx_v[b], sems[b])
            pltpu.async_copy(x_hbm.at[pl.ds(ebase + g * _B, _B)],
                             rows_v[b], sems[b])

        def wait_loads(g, b):
            pltpu.make_async_copy(idx_hbm.at[pl.ds(ebase + g * _B, _B)],
                                  idx_v[b], sems[b]).wait()
            pltpu.make_async_copy(x_hbm.at[pl.ds(ebase + g * _B, _B)],
                                  rows_v[b], sems[b]).wait()

        for b in range(_RING):
            start_loads(b, b)

        zeros16 = jnp.zeros((16,), jnp.float32)

        def zero_row(i, _):
            for j in range(d // 16):
                zero_v[i, pl.ds(j * 16, 16)] = zeros16
            return 0

        lax.fori_loop(0, _ZB, zero_row, 0)

        zbase = sid * rows_per_tile
        full, rem = divmod(rows_per_tile, _ZB)
        zcopies = []
        for k in range(full):
            zcopies.append(pltpu.async_copy(
                zero_v, acc_sh.at[pl.ds(zbase + k * _ZB, _ZB)], sem_z))
        if rem:
            zcopies.append(pltpu.async_copy(
                zero_v.at[pl.ds(0, rem)],
                acc_sh.at[pl.ds(zbase + full * _ZB, rem)], sem_z))

        @pl.when(sid == 0)
        def _zero_tail():
            pltpu.async_copy(
                zero_v.at[pl.ds(0, rows_rem)],
                acc_sh.at[pl.ds(rows_per_tile * _NS, rows_rem)],
                sem_z).wait()

        for c in zcopies:
            c.wait()
        plsc.subcore_barrier()

        @pl.loop(0, n_iter - 1, step=_RING)
        def _ring(i):
            for b in range(_RING):
                g = i + b
                wait_loads(g, b)
                pltpu.sync_copy(rows_v[b], acc_sh.at[idx_v[b]], add=True)

                @pl.when(g + _RING < n_iter)
                def _prefetch():
                    start_loads(g + _RING, b)

        g_last = n_iter - 1
        wait_loads(g_last, 0)
        pltpu.sync_copy(rows_v[0], acc_sh.at[idx_v[0]], add=True)
        plsc.subcore_barrier()

        obase = sid * rows_per_tile
        pltpu.sync_copy(
            acc_sh.at[pl.ds(obase, rows_per_tile)],
            out_hbm.at[cid, pl.ds(obase, rows_per_tile)],
        )

        @pl.when(sid == 0)
        def _write_tail():
            pltpu.sync_copy(
                acc_sh.at[pl.ds(rows_per_tile * _NS, rows_rem)],
                out_hbm.at[cid, pl.ds(rows_per_tile * _NS, rows_rem)],
            )

    return scatter_kernel(x, idx)


def _combine(p_ref, o_ref):
    o_ref[...] = (p_ref[0] + p_ref[1]) * _SCALE


def kernel(x, index):
    idx = index.astype(jnp.int32)
    partials = _sc_scatter_partials(x, idx)
    n, d = _N, x.shape[1]
    blk = 5000
    out = pl.pallas_call(
        _combine,
        grid=(n // blk,),
        in_specs=[pl.BlockSpec((_NC, blk, d), lambda i: (0, i, 0))],
        out_specs=pl.BlockSpec((blk, d), lambda i: (i, 0)),
        out_shape=jax.ShapeDtypeStruct((n, d), jnp.float32),
    )(partials)
    return out
